# Initial kernel scaffold; baseline (speedup 1.0000x reference)
import jax
import jax.numpy as jnp
from jax.experimental import pallas as pl


def kernel(graph_embedding, edge_index, W1, b0, b1):
    raise NotImplementedError("write your pallas kernel here")


# speedup vs baseline: 6.6849x; 6.6849x over previous
import functools

import jax
import jax.numpy as jnp
from jax import lax
from jax.experimental import pallas as pl
from jax.experimental.pallas import tpu as pltpu
from jax.experimental.pallas import tpu_sc as plsc

NC = 2
NS = 16
NW = NC * NS
LN = 16
K = 80


def _degree_body(srcr, dstr, hout, hin, sbuf, dbuf, ho, hi):
    c = lax.axis_index("c")
    s = lax.axis_index("s")
    wid = c * NS + s
    npad = ho.shape[0]
    zeros = jnp.zeros((LN,), jnp.float32)
    ones = jnp.ones((LN,), jnp.float32)

    def zbody(i, carry):
        ho[pl.ds(i * LN, LN)] = zeros
        hi[pl.ds(i * LN, LN)] = zeros
        return carry

    lax.fori_loop(0, npad // LN, zbody, 0)
    pltpu.sync_copy(srcr.at[wid], sbuf)
    pltpu.sync_copy(dstr.at[wid], dbuf)
    nchunk = sbuf.shape[0]

    def ebody(i, carry):
        for k in range(K // LN):
            si = sbuf[i, pl.ds(k * LN, LN)]
            di = dbuf[i, pl.ds(k * LN, LN)]
            plsc.addupdate_scatter(ho, [si], ones)
            plsc.addupdate_scatter(hi, [di], ones)
        return carry

    lax.fori_loop(0, nchunk, ebody, 0)
    pltpu.sync_copy(ho, hout.at[wid])
    pltpu.sync_copy(hi, hin.at[wid])


def _spmm_body(x, srcr, dstr, out, sbuf, dbuf, gbuf, acc):
    c = lax.axis_index("c")
    s = lax.axis_index("s")
    wid = c * NS + s
    d = gbuf.shape[1]
    zeros = jnp.zeros((LN,), jnp.float32)

    def zbody(i, carry):
        r = i // (d // LN)
        col = (i % (d // LN)) * LN
        gbuf[r, pl.ds(col, LN)] = zeros
        return carry

    lax.fori_loop(0, K * d // LN, zbody, 0)
    rows_per_tile = acc.shape[0] // NS
    base = s * rows_per_tile
    for j in range(rows_per_tile // K):
        pltpu.sync_copy(gbuf, acc.at[pl.ds(base + j * K, K)])
    pltpu.sync_copy(srcr.at[wid], sbuf)
    pltpu.sync_copy(dstr.at[wid], dbuf)
    plsc.subcore_barrier()

    nchunk = sbuf.shape[0]

    def ebody(i, carry):
        pltpu.sync_copy(x.at[sbuf.at[i]], gbuf)
        pltpu.sync_copy(gbuf, acc.at[dbuf.at[i]], add=True)
        return carry

    lax.fori_loop(0, nchunk, ebody, 0)
    plsc.subcore_barrier()
    for j in range(rows_per_tile // K):
        sl = pl.ds(base + j * K, K)
        pltpu.sync_copy(acc.at[sl], gbuf)
        pltpu.sync_copy(gbuf, out.at[c, sl])


def _scales_body(hout_ref, hin_ref, a_ref, g_ref, b_ref, raw_ref):
    dout = jnp.sum(hout_ref[...], axis=0, keepdims=True)
    din = jnp.sum(hin_ref[...], axis=0, keepdims=True)
    a = lax.rsqrt(jnp.maximum(dout, 1.0))
    b = lax.rsqrt(jnp.maximum(din, 1.0))
    a_ref[...] = a
    g_ref[...] = a * b
    b_ref[...] = b
    raw_ref[...] = din


def _rowscale_body(x_ref, s_ref, o_ref):
    o_ref[...] = x_ref[...] * s_ref[...]


def _combine_body(p0_ref, p1_ref, s_ref, a_ref, brow_ref, o_ref):
    o_ref[...] = (p0_ref[0] + p1_ref[0]) * s_ref[...] + a_ref[...] * brow_ref[...]


def _matmul_body(p0_ref, p1_ref, w_ref, raw_ref, brow_ref, o_ref):
    h = p0_ref[0] + p1_ref[0]
    o_ref[...] = (
        jnp.dot(h, w_ref[...], preferred_element_type=jnp.float32,
                precision=lax.Precision.HIGHEST)
        + raw_ref[...] * brow_ref[...]
    )


@functools.lru_cache(maxsize=None)
def _build(n, e, d):
    npad = ((n + NW * K - 1) // (NW * K)) * (NW * K)
    ep = e // NW
    nchunk = ep // K
    mesh = plsc.VectorSubcoreMesh(core_axis_name="c", subcore_axis_name="s")

    sc_params = pltpu.CompilerParams(needs_layout_passes=False)

    degree = pl.kernel(
        _degree_body,
        out_type=[jax.ShapeDtypeStruct((NW, npad), jnp.float32)] * 2,
        mesh=mesh,
        compiler_params=sc_params,
        scratch_types=[
            pltpu.VMEM((nchunk, K), jnp.int32),
            pltpu.VMEM((nchunk, K), jnp.int32),
            pltpu.VMEM((npad,), jnp.float32),
            pltpu.VMEM((npad,), jnp.float32),
        ],
    )

    spmm = pl.kernel(
        _spmm_body,
        out_type=jax.ShapeDtypeStruct((NC, npad, d), jnp.float32),
        mesh=mesh,
        compiler_params=sc_params,
        scratch_types=[
            pltpu.VMEM((nchunk, K), jnp.int32),
            pltpu.VMEM((nchunk, K), jnp.int32),
            pltpu.VMEM((K, d), jnp.float32),
            pltpu.VMEM_SHARED((npad, d), jnp.float32),
        ],
    )

    scales = pl.pallas_call(
        _scales_body,
        out_shape=[jax.ShapeDtypeStruct((1, npad), jnp.float32)] * 4,
    )

    R = 400
    grid = (n // R,)
    vec_spec = pl.BlockSpec((R, 1), lambda i: (i, 0))
    row_spec = pl.BlockSpec((R, d), lambda i: (i, 0))
    part0_spec = pl.BlockSpec((1, R, d), lambda i: (0, i, 0))
    part1_spec = pl.BlockSpec((1, R, d), lambda i: (1, i, 0))
    brow_spec = pl.BlockSpec((1, d), lambda i: (0, 0))

    rowscale = pl.pallas_call(
        _rowscale_body,
        grid=grid,
        in_specs=[row_spec, vec_spec],
        out_specs=row_spec,
        out_shape=jax.ShapeDtypeStruct((npad, d), jnp.float32),
    )

    combine = pl.pallas_call(
        _combine_body,
        grid=grid,
        in_specs=[part0_spec, part1_spec, vec_spec, vec_spec, brow_spec],
        out_specs=row_spec,
        out_shape=jax.ShapeDtypeStruct((npad, d), jnp.float32),
    )

    matmul = pl.pallas_call(
        _matmul_body,
        grid=grid,
        in_specs=[part0_spec, part1_spec,
                  pl.BlockSpec((d, d), lambda i: (0, 0)),
                  vec_spec, brow_spec],
        out_specs=row_spec,
        out_shape=jax.ShapeDtypeStruct((n, d), jnp.float32),
    )

    return degree, spmm, scales, rowscale, combine, matmul


def kernel(graph_embedding, edge_index, W1, b0, b1):
    x = graph_embedding
    n, d = x.shape
    e = edge_index.shape[1]
    degree, spmm, scales, rowscale, combine, matmul = _build(n, e, d)
    npad = ((n + NW * K - 1) // (NW * K)) * (NW * K)
    ep = e // NW

    srcr = edge_index[0].reshape(NW, ep // K, K)
    dstr = edge_index[1].reshape(NW, ep // K, K)

    hout, hin = degree(srcr, dstr)
    a, g, b, raw = scales(hout, hin)
    acol = a.reshape(npad, 1)
    gcol = g.reshape(npad, 1)
    bcol = b.reshape(npad, 1)
    rawcol = raw.reshape(npad, 1)
    zrow = jnp.zeros((1, d), jnp.float32)

    x0 = rowscale(x, acol[:n])
    p = spmm(x0, srcr, dstr)
    x1 = combine(p, p, gcol, acol, b0[None, :].astype(jnp.float32))
    q = spmm(x1, srcr, dstr)
    x2 = combine(q, q, bcol, acol, zrow)
    r = spmm(x2, srcr, dstr)
    out = matmul(r, r, W1, rawcol, b1[None, :].astype(jnp.float32))
    return out


# speedup vs baseline: 8.2452x; 1.2334x over previous
import functools

import jax
import jax.numpy as jnp
from jax import lax
from jax.experimental import pallas as pl
from jax.experimental.pallas import tpu as pltpu
from jax.experimental.pallas import tpu_sc as plsc

NC = 2
NS = 16
NW = NC * NS
LN = 16
K = 80


def _degree_body(er, hout, hin, ebuf, ho, hi):
    c = lax.axis_index("c")
    s = lax.axis_index("s")
    wid = c * NS + s
    npad = ho.shape[0]
    zeros = jnp.zeros((LN,), jnp.float32)
    ones = jnp.ones((LN,), jnp.float32)

    def zbody(i, carry):
        ho[pl.ds(i * LN, LN)] = zeros
        hi[pl.ds(i * LN, LN)] = zeros
        return carry

    lax.fori_loop(0, npad // LN, zbody, 0)
    pltpu.sync_copy(er.at[wid], ebuf)
    nchunk = ebuf.shape[0]

    def ebody(i, carry):
        for k in range(K // LN):
            si = ebuf[i, 0, pl.ds(k * LN, LN)]
            di = ebuf[i, 1, pl.ds(k * LN, LN)]
            plsc.addupdate_scatter(ho, [si], ones)
            plsc.addupdate_scatter(hi, [di], ones)
        return carry

    lax.fori_loop(0, nchunk, ebody, 0)
    pltpu.sync_copy(ho, hout.at[wid])
    pltpu.sync_copy(hi, hin.at[wid])


def _spmm_body(x, er, out, ibuf, gbuf, acc, sem_i, sem_g, sem_s):
    cc = lax.axis_index("c")
    s = lax.axis_index("s")
    wid = cc * NS + s
    d = gbuf.shape[2]
    nchunk = er.shape[1]
    zeros = jnp.zeros((LN,), jnp.float32)

    def zbody(i, carry):
        r = i // (d // LN)
        col = (i % (d // LN)) * LN
        gbuf[0, r, pl.ds(col, LN)] = zeros
        return carry

    lax.fori_loop(0, K * d // LN, zbody, 0)
    rows_per_tile = acc.shape[0] // NS
    base = s * rows_per_tile
    for j in range(rows_per_tile // K):
        pltpu.sync_copy(gbuf.at[0], acc.at[pl.ds(base + j * K, K)])
    plsc.subcore_barrier()

    pltpu.sync_copy(er.at[wid, 0], ibuf.at[0])
    pltpu.async_copy(x.at[ibuf.at[0, 0]], gbuf.at[0], sem_g)
    pltpu.async_copy(er.at[wid, 1], ibuf.at[1], sem_i)

    def ebody(c, carry):
        i3 = lax.rem(c, 3)
        p3 = lax.rem(c + 2, 3)
        n3 = lax.rem(c + 1, 3)
        g2 = lax.rem(c, 2)
        n2 = lax.rem(c + 1, 2)
        pltpu.make_async_copy(x.at[ibuf.at[i3, 0]], gbuf.at[g2], sem_g).wait()

        @pl.when(c > 0)
        def _():
            pltpu.make_async_copy(
                gbuf.at[n2], acc.at[ibuf.at[p3, 1]], sem_s).wait()

        pltpu.async_copy(gbuf.at[g2], acc.at[ibuf.at[i3, 1]], sem_s, add=True)

        @pl.when(c + 2 < nchunk)
        def _():
            pltpu.async_copy(er.at[wid, c + 2], ibuf.at[p3], sem_i)

        @pl.when(c + 1 < nchunk)
        def _():
            pltpu.make_async_copy(er.at[wid, c + 1], ibuf.at[n3], sem_i).wait()
            pltpu.async_copy(x.at[ibuf.at[n3, 0]], gbuf.at[n2], sem_g)

        return carry

    lax.fori_loop(0, nchunk, ebody, 0)
    last3 = (nchunk - 1) % 3
    last2 = (nchunk - 1) % 2
    pltpu.make_async_copy(
        gbuf.at[last2], acc.at[ibuf.at[last3, 1]], sem_s).wait()
    plsc.subcore_barrier()
    sl = pl.ds(base, rows_per_tile)
    pltpu.sync_copy(acc.at[sl], out.at[cc, sl])


def _scales_body(hout_ref, hin_ref, a_ref, g_ref, b_ref, raw_ref):
    dout = jnp.sum(hout_ref[...], axis=0, keepdims=True)
    din = jnp.sum(hin_ref[...], axis=0, keepdims=True)
    a = lax.rsqrt(jnp.maximum(dout, 1.0))
    b = lax.rsqrt(jnp.maximum(din, 1.0))
    a_ref[...] = a
    g_ref[...] = a * b
    b_ref[...] = b
    raw_ref[...] = din


def _rowscale_body(x_ref, s_ref, o_ref):
    o_ref[...] = x_ref[...] * s_ref[...]


def _combine_body(p0_ref, p1_ref, s_ref, a_ref, brow_ref, o_ref):
    o_ref[...] = (p0_ref[0] + p1_ref[0]) * s_ref[...] + a_ref[...] * brow_ref[...]


def _matmul_body(p0_ref, p1_ref, w_ref, raw_ref, brow_ref, o_ref):
    h = p0_ref[0] + p1_ref[0]
    o_ref[...] = (
        jnp.dot(h, w_ref[...], preferred_element_type=jnp.float32,
                precision=lax.Precision.HIGHEST)
        + raw_ref[...] * brow_ref[...]
    )


@functools.lru_cache(maxsize=None)
def _build(n, e, d):
    npad = ((n + NW * K - 1) // (NW * K)) * (NW * K)
    ep = e // NW
    nchunk = ep // K
    mesh = plsc.VectorSubcoreMesh(core_axis_name="c", subcore_axis_name="s")

    sc_params = pltpu.CompilerParams(needs_layout_passes=False)

    degree = pl.kernel(
        _degree_body,
        out_type=[jax.ShapeDtypeStruct((NW, npad), jnp.float32)] * 2,
        mesh=mesh,
        compiler_params=sc_params,
        scratch_types=[
            pltpu.VMEM((nchunk, 2, K), jnp.int32),
            pltpu.VMEM((npad,), jnp.float32),
            pltpu.VMEM((npad,), jnp.float32),
        ],
    )

    spmm = pl.kernel(
        _spmm_body,
        out_type=jax.ShapeDtypeStruct((NC, npad, d), jnp.float32),
        mesh=mesh,
        compiler_params=sc_params,
        scratch_types=[
            pltpu.VMEM((3, 2, K), jnp.int32),
            pltpu.VMEM((2, K, d), jnp.float32),
            pltpu.VMEM_SHARED((npad, d), jnp.float32),
            pltpu.SemaphoreType.DMA,
            pltpu.SemaphoreType.DMA,
            pltpu.SemaphoreType.DMA,
        ],
    )

    scales = pl.pallas_call(
        _scales_body,
        out_shape=[jax.ShapeDtypeStruct((1, npad), jnp.float32)] * 4,
    )

    R = 400
    grid = (n // R,)
    vec_spec = pl.BlockSpec((R, 1), lambda i: (i, 0))
    row_spec = pl.BlockSpec((R, d), lambda i: (i, 0))
    part0_spec = pl.BlockSpec((1, R, d), lambda i: (0, i, 0))
    part1_spec = pl.BlockSpec((1, R, d), lambda i: (1, i, 0))
    brow_spec = pl.BlockSpec((1, d), lambda i: (0, 0))

    rowscale = pl.pallas_call(
        _rowscale_body,
        grid=grid,
        in_specs=[row_spec, vec_spec],
        out_specs=row_spec,
        out_shape=jax.ShapeDtypeStruct((npad, d), jnp.float32),
    )

    combine = pl.pallas_call(
        _combine_body,
        grid=grid,
        in_specs=[part0_spec, part1_spec, vec_spec, vec_spec, brow_spec],
        out_specs=row_spec,
        out_shape=jax.ShapeDtypeStruct((npad, d), jnp.float32),
    )

    matmul = pl.pallas_call(
        _matmul_body,
        grid=grid,
        in_specs=[part0_spec, part1_spec,
                  pl.BlockSpec((d, d), lambda i: (0, 0)),
                  vec_spec, brow_spec],
        out_specs=row_spec,
        out_shape=jax.ShapeDtypeStruct((n, d), jnp.float32),
    )

    return degree, spmm, scales, rowscale, combine, matmul


def kernel(graph_embedding, edge_index, W1, b0, b1):
    x = graph_embedding
    n, d = x.shape
    e = edge_index.shape[1]
    degree, spmm, scales, rowscale, combine, matmul = _build(n, e, d)
    npad = ((n + NW * K - 1) // (NW * K)) * (NW * K)
    ep = e // NW

    er = jnp.stack(
        [edge_index[0].reshape(NW, ep // K, K),
         edge_index[1].reshape(NW, ep // K, K)], axis=2)

    hout, hin = degree(er)
    a, g, b, raw = scales(hout, hin)
    acol = a.reshape(npad, 1)
    gcol = g.reshape(npad, 1)
    bcol = b.reshape(npad, 1)
    rawcol = raw.reshape(npad, 1)
    zrow = jnp.zeros((1, d), jnp.float32)

    x0 = rowscale(x, acol[:n])
    p = spmm(x0, er)
    x1 = combine(p, p, gcol, acol, b0[None, :].astype(jnp.float32))
    q = spmm(x1, er)
    x2 = combine(q, q, bcol, acol, zrow)
    r = spmm(x2, er)
    out = matmul(r, r, W1, rawcol, b1[None, :].astype(jnp.float32))
    return out


# speedup vs baseline: 11.4551x; 1.3893x over previous
import functools

import jax
import jax.numpy as jnp
from jax import lax
from jax.experimental import pallas as pl
from jax.experimental.pallas import tpu as pltpu
from jax.experimental.pallas import tpu_sc as plsc

NC = 2
NS = 16
NW = NC * NS
LN = 16
K = 80


def _degree_body(er, hout, hin, ebuf, ho, hi):
    c = lax.axis_index("c")
    s = lax.axis_index("s")
    wid = c * NS + s
    npad = ho.shape[0]
    zeros = jnp.zeros((LN,), jnp.float32)
    ones = jnp.ones((LN,), jnp.float32)

    def zbody(i, carry):
        ho[pl.ds(i * LN, LN)] = zeros
        hi[pl.ds(i * LN, LN)] = zeros
        return carry

    lax.fori_loop(0, npad // LN, zbody, 0)
    pltpu.sync_copy(er.at[wid], ebuf)
    nchunk = ebuf.shape[0]

    def ebody(i, carry):
        for k in range(K // LN):
            si = ebuf[i, 0, pl.ds(k * LN, LN)]
            di = ebuf[i, 1, pl.ds(k * LN, LN)]
            plsc.addupdate_scatter(ho, [si], ones)
            plsc.addupdate_scatter(hi, [di], ones)
        return carry

    lax.fori_loop(0, nchunk, ebody, 0)
    pltpu.sync_copy(ho, hout.at[wid])
    pltpu.sync_copy(hi, hin.at[wid])


def _spmm_body(x, er, out, ibuf, gbuf, acc, sem_i, sem_g, sem_s):
    cc = lax.axis_index("c")
    s = lax.axis_index("s")
    wid = cc * NS + s
    d = gbuf.shape[2]
    nchunk = er.shape[1]
    zeros = jnp.zeros((LN,), jnp.float32)

    def zbody(i, carry):
        r = i // (d // LN)
        col = (i % (d // LN)) * LN
        gbuf[0, r, pl.ds(col, LN)] = zeros
        return carry

    lax.fori_loop(0, K * d // LN, zbody, 0)
    rows_per_tile = acc.shape[0] // NS
    base = s * rows_per_tile
    for j in range(rows_per_tile // K):
        pltpu.sync_copy(gbuf.at[0], acc.at[pl.ds(base + j * K, K)])
    plsc.subcore_barrier()

    pltpu.sync_copy(er.at[wid, 0], ibuf.at[0])
    pltpu.async_copy(x.at[ibuf.at[0, 0]], gbuf.at[0], sem_g.at[0])
    pltpu.async_copy(er.at[wid, 1], ibuf.at[1], sem_i.at[1])
    pltpu.async_copy(er.at[wid, 2], ibuf.at[2], sem_i.at[0])
    pltpu.make_async_copy(er.at[wid, 1], ibuf.at[1], sem_i.at[1]).wait()
    pltpu.async_copy(x.at[ibuf.at[1, 0]], gbuf.at[1], sem_g.at[1])

    def ebody(c, carry):
        par = lax.rem(c, 2)
        i3 = lax.rem(c, 3)
        i4 = lax.rem(c, 4)
        p3 = lax.rem(c + 2, 3)
        p4 = lax.rem(c + 3, 4)
        n3 = lax.rem(c + 2, 3)
        n4 = lax.rem(c + 2, 4)
        npar = lax.rem(c + 1, 2)
        pltpu.make_async_copy(
            x.at[ibuf.at[i4, 0]], gbuf.at[i3], sem_g.at[par]).wait()

        @pl.when(c > 0)
        def _():
            pltpu.make_async_copy(
                gbuf.at[p3], acc.at[ibuf.at[p4, 1]], sem_s).wait()

        pltpu.async_copy(gbuf.at[i3], acc.at[ibuf.at[i4, 1]], sem_s, add=True)

        @pl.when(c + 3 < nchunk)
        def _():
            pltpu.async_copy(er.at[wid, c + 3], ibuf.at[p4], sem_i.at[npar])

        @pl.when(c + 2 < nchunk)
        def _():
            pltpu.make_async_copy(
                er.at[wid, c + 2], ibuf.at[n4], sem_i.at[par]).wait()
            pltpu.async_copy(x.at[ibuf.at[n4, 0]], gbuf.at[n3], sem_g.at[par])

        return carry

    lax.fori_loop(0, nchunk, ebody, 0)
    last3 = (nchunk - 1) % 3
    last4 = (nchunk - 1) % 4
    pltpu.make_async_copy(
        gbuf.at[last3], acc.at[ibuf.at[last4, 1]], sem_s).wait()
    del last3, last4
    plsc.subcore_barrier()
    sl = pl.ds(base, rows_per_tile)
    pltpu.sync_copy(acc.at[sl], out.at[cc, sl])


def _scales_body(hout_ref, hin_ref, a_ref, g_ref, b_ref, raw_ref):
    dout = jnp.sum(hout_ref[...], axis=0, keepdims=True)
    din = jnp.sum(hin_ref[...], axis=0, keepdims=True)
    a = lax.rsqrt(jnp.maximum(dout, 1.0))
    b = lax.rsqrt(jnp.maximum(din, 1.0))
    a_ref[...] = a
    g_ref[...] = a * b
    b_ref[...] = b
    raw_ref[...] = din


def _rowscale_body(x_ref, s_ref, o_ref):
    o_ref[...] = x_ref[...] * s_ref[...]


def _combine_body(p0_ref, p1_ref, s_ref, a_ref, brow_ref, o_ref):
    o_ref[...] = (p0_ref[0] + p1_ref[0]) * s_ref[...] + a_ref[...] * brow_ref[...]


def _matmul_body(p0_ref, p1_ref, w_ref, raw_ref, brow_ref, o_ref):
    h = p0_ref[0] + p1_ref[0]
    o_ref[...] = (
        jnp.dot(h, w_ref[...], preferred_element_type=jnp.float32,
                precision=lax.Precision.HIGHEST)
        + raw_ref[...] * brow_ref[...]
    )


@functools.lru_cache(maxsize=None)
def _build(n, e, d):
    npad = ((n + NW * K - 1) // (NW * K)) * (NW * K)
    ep = e // NW
    nchunk = ep // K
    mesh = plsc.VectorSubcoreMesh(core_axis_name="c", subcore_axis_name="s")

    sc_params = pltpu.CompilerParams(needs_layout_passes=False)

    degree = pl.kernel(
        _degree_body,
        out_type=[jax.ShapeDtypeStruct((NW, npad), jnp.float32)] * 2,
        mesh=mesh,
        compiler_params=sc_params,
        scratch_types=[
            pltpu.VMEM((nchunk, 2, K), jnp.int32),
            pltpu.VMEM((npad,), jnp.float32),
            pltpu.VMEM((npad,), jnp.float32),
        ],
    )

    spmm = pl.kernel(
        _spmm_body,
        out_type=jax.ShapeDtypeStruct((NC, npad, d), jnp.float32),
        mesh=mesh,
        compiler_params=sc_params,
        scratch_types=[
            pltpu.VMEM((4, 2, K), jnp.int32),
            pltpu.VMEM((3, K, d), jnp.float32),
            pltpu.VMEM_SHARED((npad, d), jnp.float32),
            pltpu.SemaphoreType.DMA((2,)),
            pltpu.SemaphoreType.DMA((2,)),
            pltpu.SemaphoreType.DMA,
        ],
    )

    scales = pl.pallas_call(
        _scales_body,
        out_shape=[jax.ShapeDtypeStruct((1, npad), jnp.float32)] * 4,
    )

    R = 400
    grid = (n // R,)
    vec_spec = pl.BlockSpec((R, 1), lambda i: (i, 0))
    row_spec = pl.BlockSpec((R, d), lambda i: (i, 0))
    part0_spec = pl.BlockSpec((1, R, d), lambda i: (0, i, 0))
    part1_spec = pl.BlockSpec((1, R, d), lambda i: (1, i, 0))
    brow_spec = pl.BlockSpec((1, d), lambda i: (0, 0))

    rowscale = pl.pallas_call(
        _rowscale_body,
        grid=grid,
        in_specs=[row_spec, vec_spec],
        out_specs=row_spec,
        out_shape=jax.ShapeDtypeStruct((npad, d), jnp.float32),
    )

    combine = pl.pallas_call(
        _combine_body,
        grid=grid,
        in_specs=[part0_spec, part1_spec, vec_spec, vec_spec, brow_spec],
        out_specs=row_spec,
        out_shape=jax.ShapeDtypeStruct((npad, d), jnp.float32),
    )

    matmul = pl.pallas_call(
        _matmul_body,
        grid=grid,
        in_specs=[part0_spec, part1_spec,
                  pl.BlockSpec((d, d), lambda i: (0, 0)),
                  vec_spec, brow_spec],
        out_specs=row_spec,
        out_shape=jax.ShapeDtypeStruct((n, d), jnp.float32),
    )

    return degree, spmm, scales, rowscale, combine, matmul


def kernel(graph_embedding, edge_index, W1, b0, b1):
    x = graph_embedding
    n, d = x.shape
    e = edge_index.shape[1]
    degree, spmm, scales, rowscale, combine, matmul = _build(n, e, d)
    npad = ((n + NW * K - 1) // (NW * K)) * (NW * K)
    ep = e // NW

    er = jnp.stack(
        [edge_index[0].reshape(NW, ep // K, K),
         edge_index[1].reshape(NW, ep // K, K)], axis=2)

    hout, hin = degree(er)
    a, g, b, raw = scales(hout, hin)
    acol = a.reshape(npad, 1)
    gcol = g.reshape(npad, 1)
    bcol = b.reshape(npad, 1)
    rawcol = raw.reshape(npad, 1)
    zrow = jnp.zeros((1, d), jnp.float32)

    x0 = rowscale(x, acol[:n])
    p = spmm(x0, er)
    x1 = combine(p, p, gcol, acol, b0[None, :].astype(jnp.float32))
    q = spmm(x1, er)
    x2 = combine(q, q, bcol, acol, zrow)
    r = spmm(x2, er)
    out = matmul(r, r, W1, rawcol, b1[None, :].astype(jnp.float32))
    return out
